# baseline (speedup 1.0000x reference)
import functools

import jax
import jax.numpy as jnp
from jax import lax
from jax.experimental import pallas as pl
from jax.experimental.pallas import tpu as pltpu
from jax.experimental.pallas import tpu_sc as plsc

N = 6_400_000
NUM_CORES = 2
NUM_SUBCORES = 16
NW = NUM_CORES * NUM_SUBCORES
T = N // NW
S = 10_000
R = T // S
NBUF = 4


def kernel(edge_weights, edge_index):
    mesh = plsc.VectorSubcoreMesh(
        core_axis_name="c", subcore_axis_name="s",
        num_cores=NUM_CORES, num_subcores=NUM_SUBCORES,
    )

    @functools.partial(
        pl.kernel,
        mesh=mesh,
        out_type=jax.ShapeDtypeStruct((N,), jnp.float32),
        scratch_types=(
            [pltpu.VMEM((S,), jnp.int32) for _ in range(NBUF)]
            + [pltpu.VMEM((S,), jnp.float32) for _ in range(NBUF)]
            + [pltpu.SemaphoreType.DMA for _ in range(4 * NBUF)]
        ),
    )
    def gather_kernel(w_hbm, idx_hbm, out_hbm, *scratch):
        ix = scratch[0:NBUF]
        vv = scratch[NBUF:2 * NBUF]
        s_in = scratch[2 * NBUF:3 * NBUF]
        s_g = scratch[3 * NBUF:4 * NBUF]
        s_o = scratch[4 * NBUF:5 * NBUF]
        s_g2 = scratch[5 * NBUF:6 * NBUF]

        wid = lax.axis_index("s") * NUM_CORES + lax.axis_index("c")
        base = wid * T

        H1 = 5_120
        H2 = S - H1

        class _GPair:
            def __init__(self, b):
                self.c1 = pltpu.make_async_copy(
                    w_hbm.at[ix[b].at[pl.ds(0, H1)]],
                    vv[b].at[pl.ds(0, H1)], s_g[b])
                self.c2 = pltpu.make_async_copy(
                    w_hbm.at[ix[b].at[pl.ds(H1, H2)]],
                    vv[b].at[pl.ds(H1, H2)], s_g2[b])

            def start(self):
                self.c1.start()
                self.c2.start()

            def wait(self):
                self.c1.wait()
                self.c2.wait()

        def g_copy(b):
            return _GPair(b)

        def out_copy(r, b):
            return pltpu.make_async_copy(
                vv[b], out_hbm.at[pl.ds(base + r * S, S)], s_o[b])

        def in_copy_d(r, b):
            return pltpu.make_async_copy(
                idx_hbm.at[pl.ds(base + r * S, S)], ix[b], s_in[b])

        def round_step(r, b, *, drain_out, wait_prev_g, prefetch):
            if drain_out:
                out_copy(r - NBUF, b).wait()
            in_copy_d(r, b).wait()
            g_copy(b).start()
            if wait_prev_g:
                pb = (b - 1) % NBUF
                g_copy(pb).wait()
                out_copy(r - 1, pb).start()
            if prefetch:
                nb = (b - 1) % NBUF
                in_copy_d(r + NBUF - 1, nb).start()

        for r in range(NBUF - 1):
            in_copy_d(r, r % NBUF).start()
        round_step(0, 0, drain_out=False, wait_prev_g=False, prefetch=True)
        for r in range(1, NBUF):
            round_step(r, r % NBUF, drain_out=False, wait_prev_g=True,
                       prefetch=True)

        def steady(i, carry):
            r0 = NBUF + i * NBUF
            for b in range(NBUF):
                round_step(r0 + b, b, drain_out=True, wait_prev_g=True,
                           prefetch=True)
            return carry

        n_steady = (R - NBUF) // NBUF - 1
        lax.fori_loop(0, n_steady, steady, 0)

        r0 = NBUF + n_steady * NBUF
        for b in range(NBUF):
            round_step(r0 + b, b, drain_out=True, wait_prev_g=True,
                       prefetch=(r0 + b + NBUF - 1 < R))

        r0 += NBUF
        for b in range(R - r0):
            round_step(r0 + b, b, drain_out=True, wait_prev_g=True,
                       prefetch=False)
        last_b = (R - 1) % NBUF
        g_copy(last_b).wait()
        out_copy(R - 1, last_b).start()
        for k in range(NBUF):
            out_copy(R - NBUF + k, (R - NBUF + k) % NBUF).wait()

    return gather_kernel(edge_weights, edge_index)
